# baseline (speedup 1.0000x reference)
import jax
import jax.numpy as jnp
from jax import lax
from jax.experimental import pallas as pl
from jax.experimental.pallas import tpu as pltpu

_BB = 2
_NSLOT = 8
_DEPTH = 5


def kernel(patch, pos_table):
    B, P, D = patch.shape
    steps = B // _BB

    def body(patch_hbm, pos_hbm, out_hbm, pos_v, *rest):
        bufs = rest[:_NSLOT]
        isems = rest[_NSLOT]
        osems = rest[_NSLOT + 1]

        cp = pltpu.make_async_copy(pos_hbm, pos_v, isems.at[0])
        cp.start()
        cp.wait()

        def start_in(j, s):
            pltpu.make_async_copy(
                patch_hbm.at[pl.ds(j * _BB, _BB)], bufs[s], isems.at[s]
            ).start()

        def wait_in(j, s):
            pltpu.make_async_copy(
                patch_hbm.at[pl.ds(j * _BB, _BB)], bufs[s], isems.at[s]
            ).wait()

        def start_out(j, s):
            pltpu.make_async_copy(
                bufs[s], out_hbm.at[pl.ds(j * _BB, _BB)], osems.at[s]
            ).start()

        def wait_out(j, s):
            pltpu.make_async_copy(
                bufs[s], out_hbm.at[pl.ds(j * _BB, _BB)], osems.at[s]
            ).wait()

        for j in range(_DEPTH):
            start_in(j, j % _NSLOT)

        def ring_body(k, carry):
            for s in range(_NSLOT):
                j = k * _NSLOT + s
                wait_in(j, s)
                bufs[s][...] = bufs[s][...] + pos_v[...][None]
                start_out(j, s)
                jn = j + _DEPTH
                sn = (s + _DEPTH) % _NSLOT

                @pl.when(jn < steps)
                def _():
                    @pl.when(jn - _NSLOT >= 0)
                    def _():
                        wait_out(jn - _NSLOT, sn)

                    start_in(jn, sn)
            return carry

        lax.fori_loop(0, steps // _NSLOT, ring_body, 0)
        for j in range(steps - _NSLOT, steps):
            wait_out(j, j % _NSLOT)

    return pl.pallas_call(
        body,
        in_specs=[
            pl.BlockSpec(memory_space=pl.ANY),
            pl.BlockSpec(memory_space=pl.ANY),
        ],
        out_specs=pl.BlockSpec(memory_space=pl.ANY),
        out_shape=jax.ShapeDtypeStruct(patch.shape, patch.dtype),
        scratch_shapes=(
            [pltpu.VMEM((P, D), jnp.float32)]
            + [pltpu.VMEM((_BB, P, D), jnp.float32) for _ in range(_NSLOT)]
            + [pltpu.SemaphoreType.DMA((_NSLOT,)),
               pltpu.SemaphoreType.DMA((_NSLOT,))]
        ),
        compiler_params=pltpu.CompilerParams(
            vmem_limit_bytes=110 * 1024 * 1024),
    )(patch, pos_table)


# speedup vs baseline: 1.0013x; 1.0013x over previous
import jax
import jax.numpy as jnp
from jax import lax
from jax.experimental import pallas as pl
from jax.experimental.pallas import tpu as pltpu

_BB = 4
_NSLOT = 4
_DEPTH = 2


def kernel(patch, pos_table):
    B, P, D = patch.shape
    steps = B // _BB

    def body(patch_hbm, pos_hbm, out_hbm, pos_v, *rest):
        bufs = rest[:_NSLOT]
        isems = rest[_NSLOT]
        osems = rest[_NSLOT + 1]

        cp = pltpu.make_async_copy(pos_hbm, pos_v, isems.at[0])
        cp.start()
        cp.wait()

        def start_in(j, s):
            pltpu.make_async_copy(
                patch_hbm.at[pl.ds(j * _BB, _BB)], bufs[s], isems.at[s]
            ).start()

        def wait_in(j, s):
            pltpu.make_async_copy(
                patch_hbm.at[pl.ds(j * _BB, _BB)], bufs[s], isems.at[s]
            ).wait()

        def start_out(j, s):
            pltpu.make_async_copy(
                bufs[s], out_hbm.at[pl.ds(j * _BB, _BB)], osems.at[s]
            ).start()

        def wait_out(j, s):
            pltpu.make_async_copy(
                bufs[s], out_hbm.at[pl.ds(j * _BB, _BB)], osems.at[s]
            ).wait()

        for j in range(_DEPTH):
            start_in(j, j % _NSLOT)

        def ring_body(k, carry):
            for s in range(_NSLOT):
                j = k * _NSLOT + s
                wait_in(j, s)
                bufs[s][...] = bufs[s][...] + pos_v[...][None]
                start_out(j, s)
                jn = j + _DEPTH
                sn = (s + _DEPTH) % _NSLOT

                @pl.when(jn < steps)
                def _():
                    @pl.when(jn - _NSLOT >= 0)
                    def _():
                        wait_out(jn - _NSLOT, sn)

                    start_in(jn, sn)
            return carry

        lax.fori_loop(0, steps // _NSLOT, ring_body, 0)
        for j in range(steps - _NSLOT, steps):
            wait_out(j, j % _NSLOT)

    return pl.pallas_call(
        body,
        in_specs=[
            pl.BlockSpec(memory_space=pl.ANY),
            pl.BlockSpec(memory_space=pl.ANY),
        ],
        out_specs=pl.BlockSpec(memory_space=pl.ANY),
        out_shape=jax.ShapeDtypeStruct(patch.shape, patch.dtype),
        scratch_shapes=(
            [pltpu.VMEM((P, D), jnp.float32)]
            + [pltpu.VMEM((_BB, P, D), jnp.float32) for _ in range(_NSLOT)]
            + [pltpu.SemaphoreType.DMA((_NSLOT,)),
               pltpu.SemaphoreType.DMA((_NSLOT,))]
        ),
        compiler_params=pltpu.CompilerParams(
            vmem_limit_bytes=110 * 1024 * 1024),
    )(patch, pos_table)


# speedup vs baseline: 1.0097x; 1.0084x over previous
import jax
import jax.numpy as jnp
from jax.experimental import pallas as pl
from jax.experimental.pallas import tpu as pltpu


def _add_body(patch_ref, pos_ref, out_ref):
    out_ref[...] = patch_ref[...] + pos_ref[...]


def kernel(patch, pos_table):
    B, P, D = patch.shape
    return pl.pallas_call(
        _add_body,
        grid=(B // 4,),
        in_specs=[
            pl.BlockSpec((4, P, D), lambda b: (b, 0, 0)),
            pl.BlockSpec((P, D), lambda b: (0, 0)),
        ],
        out_specs=pl.BlockSpec((4, P, D), lambda b: (b, 0, 0)),
        out_shape=jax.ShapeDtypeStruct(patch.shape, patch.dtype),
        compiler_params=pltpu.CompilerParams(vmem_limit_bytes=128 * 1024 * 1024),
    )(patch, pos_table)
